# baseline (speedup 1.0000x reference)
import functools

import jax
import jax.numpy as jnp
from jax import lax
from jax.experimental import pallas as pl
from jax.experimental.pallas import tpu as pltpu
from jax.experimental.pallas import tpu_sc as plsc

N_ITEMS = 1000000
LATENT_DIM = 32
MARGIN = 0.5
N_NEG = 20
BATCH = 16384

NC = 2
NS = 16
L = 16
NW = NC * NS
TB = BATCH // NW
C = 64
NCHUNK = TB // C
KC = C * N_NEG
KR = KC // 128
NHALF = N_NEG // 2


def _rsqrt(x):
    i = plsc.bitcast(x, jnp.int32)
    i = jnp.int32(0x5F3759DF) - lax.shift_right_logical(i, 1)
    y = plsc.bitcast(i, jnp.float32)
    for _ in range(3):
        t = (0.5 * x) * y
        t = t * y
        y = y * (1.5 - t)
    return y


def _clip_scale(sq):
    return jnp.minimum(1.0, _rsqrt(jnp.maximum(sq, 1e-14)))


def _sc_body(i_hbm, j_hbm, kf_hbm, ut_hbm, it_hbm, wt_hbm, out_hbm,
             iidx_v, jidx_v, kidx_v,
             urows0, irows0, nrows0,
             urows1, irows1, nrows1,
             wtab_v, loss_v, sem0, sem1):
    cid = lax.axis_index("c")
    sid = lax.axis_index("s")
    wid = sid * NC + cid
    base = wid * TB

    pltpu.sync_copy(wt_hbm, wtab_v)
    pltpu.sync_copy(i_hbm.at[pl.ds(base, TB)], iidx_v)
    pltpu.sync_copy(j_hbm.at[pl.ds(base, TB)], jidx_v)
    pltpu.sync_copy(kf_hbm.at[pl.ds(base * N_NEG, TB * N_NEG)], kidx_v)

    iota = lax.iota(jnp.int32, L)
    zerof = jnp.zeros((L,), jnp.float32)
    zeroi = jnp.zeros((L,), jnp.int32)

    bufs = ((urows0, irows0, nrows0, sem0),
            (urows1, irows1, nrows1, sem1))

    def fire(c, p):
        urows, irows, nrows, sem = bufs[p]
        pltpu.async_copy(ut_hbm.at[iidx_v.at[pl.ds(c * C, C)]], urows, sem)
        pltpu.async_copy(it_hbm.at[jidx_v.at[pl.ds(c * C, C)]], irows, sem)
        pltpu.async_copy(it_hbm.at[kidx_v.at[pl.ds(c * KC, KC)]], nrows, sem)

    def drain(c, p):
        urows, irows, nrows, sem = bufs[p]
        pltpu.make_async_copy(ut_hbm.at[iidx_v.at[pl.ds(c * C, C)]],
                              urows, sem).wait()
        pltpu.make_async_copy(it_hbm.at[jidx_v.at[pl.ds(c * C, C)]],
                              irows, sem).wait()
        pltpu.make_async_copy(it_hbm.at[kidx_v.at[pl.ds(c * KC, KC)]],
                              nrows, sem).wait()

    def group_body(g, acc, p):
        urows, irows, nrows = bufs[p][0], bufs[p][1], bufs[p][2]
        rowv = g * L + iota

        @pl.loop(0, LATENT_DIM, init_carry=(iota, zerof, zerof, zerof))
        def pass1(d, carry):
            colv, u2, i2, ui = carry
            u = plsc.load_gather(urows, [rowv, colv])
            it = plsc.load_gather(irows, [rowv, colv])
            return ((colv + 1) & (LATENT_DIM - 1),
                    u2 + u * u, i2 + it * it, ui + u * it)

        _, u2, i2, ui = pass1
        su = _clip_scale(u2)
        si = _clip_scale(i2)
        a = MARGIN + si * si * i2 - 2.0 * su * si * ui
        su2 = 2.0 * su

        cnt = zeroi
        pr = zerof
        for h in range(2):
            nrow0 = rowv * N_NEG + h * NHALF
            init = (iota,) + (zerof,) * (2 * NHALF)

            @pl.loop(0, LATENT_DIM, init_carry=init)
            def neg_pass(d, carry):
                colv = carry[0]
                n2s = list(carry[1:1 + NHALF])
                uns = list(carry[1 + NHALF:])
                u = plsc.load_gather(urows, [rowv, colv])
                for n in range(NHALF):
                    x = plsc.load_gather(nrows, [nrow0 + n, colv])
                    n2s[n] = n2s[n] + x * x
                    uns[n] = uns[n] + u * x
                return ((colv + 1) & (LATENT_DIM - 1),) + tuple(n2s) + tuple(uns)

            n2s = neg_pass[1:1 + NHALF]
            uns = neg_pass[1 + NHALF:]
            for n in range(NHALF):
                sn = _clip_scale(n2s[n])
                m = a - sn * (sn * n2s[n] - su2 * uns[n])
                pos = m > 0.0
                cnt = cnt + jnp.where(pos, 1, 0).astype(jnp.int32)
                pr = pr + jnp.where(pos, m, 0.0)

        w = plsc.load_gather(wtab_v, [cnt])
        return acc + w * pr

    def compute(acc, p):
        for g in range(C // L):
            acc = group_body(g, acc, p)
        return acc

    fire(0, 0)

    @pl.loop(0, NCHUNK, step=2, init_carry=zerof)
    def chunk_loop(c, acc):
        fire(c + 1, 1)
        drain(c, 0)
        acc = compute(acc, 0)

        @pl.when(c + 2 < NCHUNK)
        def _():
            fire(c + 2, 0)

        drain(c + 1, 1)
        return compute(acc, 1)

    loss_v[...] = chunk_loop
    pltpu.sync_copy(loss_v, out_hbm.at[pl.ds(wid * L, L)])


NROW = 1000000
SUP = 512
NSUP = NROW // SUP
NT = 122
TAIL = NROW - NSUP * SUP


def _ph1_body(utt, itt, utail, itail, uout, iout, in0, in1, in2, in3,
              ob0, ob1, semi0, semi1, semi2, semi3, semo0, semo1):
    cid = lax.axis_index("c")
    sid = lax.axis_index("s")
    wid = sid * NC + cid
    w16 = wid & 15

    iota = lax.iota(jnp.int32, L)
    inb = (in0, in1, in2, in3)
    ob = (ob0, ob1)
    semi = (semi0, semi1, semi2, semi3)
    semo = (semo0, semo1)

    def transpose_super(p, q):
        @pl.loop(0, LATENT_DIM)
        def _(c0):
            cvec = (c0 + iota) & (LATENT_DIM - 1)
            srowbase = (cvec >> 3) * 32 + (cvec & 7)
            for tj in range(4):
                srcrow = srowbase + tj * 8
                dbase = cvec + tj * 4096
                for r0 in range(8):
                    rvec = r0 * L + iota
                    v = plsc.load_gather(inb[p], [srcrow, rvec])
                    plsc.store_scatter(ob[q], [rvec * LATENT_DIM + dbase], v)

    def run_table(src, tail, dst):
        def s_of(t):
            return w16 + 16 * t

        def fire_in(t, p):
            off = pl.multiple_of(s_of(t) * SUP, SUP)
            for a in range(4):
                for tj in range(4):
                    pltpu.async_copy(
                        src.at[a, :, pl.ds(off + tj * 128, 128)],
                        inb[p].at[pl.ds((a * 4 + tj) * 8, 8)], semi[p])

        def drain_in(t, p):
            off = pl.multiple_of(s_of(t) * SUP, SUP)
            for a in range(4):
                for tj in range(4):
                    pltpu.make_async_copy(
                        src.at[a, :, pl.ds(off + tj * 128, 128)],
                        inb[p].at[pl.ds((a * 4 + tj) * 8, 8)],
                        semi[p]).wait()

        def fire_out(t, q):
            off = pl.multiple_of(s_of(t) * SUP * LATENT_DIM, SUP * LATENT_DIM)
            pltpu.async_copy(ob[q], dst.at[pl.ds(off, SUP * LATENT_DIM)],
                             semo[q])

        def drain_out(t, q):
            off = pl.multiple_of(s_of(t) * SUP * LATENT_DIM, SUP * LATENT_DIM)
            pltpu.make_async_copy(ob[q], dst.at[pl.ds(off, SUP * LATENT_DIM)],
                                  semo[q]).wait()

        def step(t, tq, p, q):
            @pl.when(tq + 3 < NT)
            def _():
                fire_in(tq + 3, (p + 3) % 4)

            drain_in(tq, p)

            @pl.when(tq >= 2)
            def _():
                drain_out(tq - 2, q)

            transpose_super(p, q)
            fire_out(tq, q)

        fire_in(0, 0)
        fire_in(1, 1)
        fire_in(2, 2)

        @pl.loop(0, NT - 2, step=4)
        def _(t):
            for qq in range(4):
                step(t, t + qq, qq, qq % 2)

        step(NT - 2, NT - 2, (NT - 2) % 4, 0)
        step(NT - 1, NT - 1, (NT - 1) % 4, 1)
        drain_out(NT - 2, 0)
        drain_out(NT - 1, 1)

        @pl.when(w16 == 0)
        def _():
            off = (NSUP - 1) * SUP
            for a in range(4):
                for tj in range(4):
                    pltpu.sync_copy(
                        src.at[a, :, pl.ds(off + tj * 128, 128)],
                        in0.at[pl.ds((a * 4 + tj) * 8, 8)])
            transpose_super(0, 0)
            pltpu.sync_copy(ob0, dst.at[pl.ds((NSUP - 1) * SUP * LATENT_DIM,
                                              SUP * LATENT_DIM)])

        @pl.when(w16 == 1)
        def _():
            pltpu.sync_copy(tail, ob0.at[pl.ds(0, TAIL * LATENT_DIM)])
            pltpu.sync_copy(ob0.at[pl.ds(0, TAIL * LATENT_DIM)],
                            dst.at[pl.ds(NSUP * SUP * LATENT_DIM,
                                         TAIL * LATENT_DIM)])

    @pl.when(wid < 16)
    def _():
        run_table(utt, utail, uout)

    @pl.when(wid >= 16)
    def _():
        run_table(itt, itail, iout)


@functools.cache
def _ph1_kernel():
    return functools.partial(
        pl.kernel,
        out_type=(jax.ShapeDtypeStruct((NROW * LATENT_DIM,), jnp.float32),
                  jax.ShapeDtypeStruct((NROW * LATENT_DIM,), jnp.float32)),
        mesh=plsc.VectorSubcoreMesh(core_axis_name="c", subcore_axis_name="s",
                                    num_cores=NC, num_subcores=NS),
        scratch_types=[
            pltpu.VMEM((128, 128), jnp.float32),
            pltpu.VMEM((128, 128), jnp.float32),
            pltpu.VMEM((128, 128), jnp.float32),
            pltpu.VMEM((128, 128), jnp.float32),
            pltpu.VMEM((SUP * LATENT_DIM,), jnp.float32),
            pltpu.VMEM((SUP * LATENT_DIM,), jnp.float32),
            pltpu.SemaphoreType.DMA,
            pltpu.SemaphoreType.DMA,
            pltpu.SemaphoreType.DMA,
            pltpu.SemaphoreType.DMA,
            pltpu.SemaphoreType.DMA,
            pltpu.SemaphoreType.DMA,
        ],
        compiler_params=pltpu.CompilerParams(needs_layout_passes=False,
                                             use_tc_tiling_on_sc=True),
    )(_ph1_body)


@functools.cache
def _sc_kernel():
    row_scratch = [pltpu.VMEM((C, LATENT_DIM), jnp.float32),
                   pltpu.VMEM((C, LATENT_DIM), jnp.float32),
                   pltpu.VMEM((KC, LATENT_DIM), jnp.float32)]
    return functools.partial(
        pl.kernel,
        out_type=jax.ShapeDtypeStruct((NW * L,), jnp.float32),
        mesh=plsc.VectorSubcoreMesh(core_axis_name="c", subcore_axis_name="s",
                                    num_cores=NC, num_subcores=NS),
        scratch_types=[
            pltpu.VMEM((TB,), jnp.int32),
            pltpu.VMEM((TB,), jnp.int32),
            pltpu.VMEM((TB * N_NEG,), jnp.int32),
        ] + row_scratch + row_scratch + [
            pltpu.VMEM((32,), jnp.float32),
            pltpu.VMEM((L,), jnp.float32),
            pltpu.SemaphoreType.DMA,
            pltpu.SemaphoreType.DMA,
        ],
        compiler_params=pltpu.CompilerParams(needs_layout_passes=False,
                                             use_tc_tiling_on_sc=False),
    )(_sc_body)


@jax.jit
def kernel(i, j, k, user_table, item_table):
    i = i.astype(jnp.int32)
    j = j.astype(jnp.int32)
    kf = k.astype(jnp.int32).reshape(BATCH * N_NEG)
    cnts = jnp.arange(N_NEG + 1, dtype=jnp.float32)
    wtab = jnp.log(N_ITEMS * cnts / N_NEG + 1.0)
    wtab = jnp.concatenate(
        [wtab, jnp.zeros((32 - (N_NEG + 1),), jnp.float32)])
    utail = user_table[NSUP * SUP:].reshape(TAIL * LATENT_DIM)
    itail = item_table[NSUP * SUP:].reshape(TAIL * LATENT_DIM)
    utt3 = user_table.T.reshape(4, 8, NROW)
    itt3 = item_table.T.reshape(4, 8, NROW)
    uflat, iflat = _ph1_kernel()(utt3, itt3, utail, itail)
    uscr = uflat.reshape(NROW, LATENT_DIM)
    iscr = iflat.reshape(NROW, LATENT_DIM)
    partial = _sc_kernel()(i, j, kf, uscr, iscr, wtab)
    return jnp.sum(partial)


# speedup vs baseline: 1.0752x; 1.0752x over previous
import functools

import jax
import jax.numpy as jnp
from jax import lax
from jax.experimental import pallas as pl
from jax.experimental.pallas import tpu as pltpu
from jax.experimental.pallas import tpu_sc as plsc

N_ITEMS = 1000000
LATENT_DIM = 32
MARGIN = 0.5
N_NEG = 20
BATCH = 16384

NC = 2
NS = 16
L = 16
NW = NC * NS
TB = BATCH // NW
C = 64
NCHUNK = TB // C
KC = C * N_NEG
KR = KC // 128
NHALF = N_NEG // 2


def _rsqrt(x):
    i = plsc.bitcast(x, jnp.int32)
    i = jnp.int32(0x5F3759DF) - lax.shift_right_logical(i, 1)
    y = plsc.bitcast(i, jnp.float32)
    for _ in range(3):
        t = (0.5 * x) * y
        t = t * y
        y = y * (1.5 - t)
    return y


def _clip_scale(sq):
    return jnp.minimum(1.0, _rsqrt(jnp.maximum(sq, 1e-14)))


def _sc_body(i_hbm, j_hbm, kf_hbm, ut_hbm, it_hbm, wt_hbm, out_hbm,
             iidx_v, jidx_v, kidx_v,
             urows0, irows0, nrows0,
             urows1, irows1, nrows1,
             wtab_v, loss_v, sem0, sem1):
    cid = lax.axis_index("c")
    sid = lax.axis_index("s")
    wid = sid * NC + cid
    base = wid * TB

    pltpu.sync_copy(wt_hbm, wtab_v)
    pltpu.sync_copy(i_hbm.at[pl.ds(base, TB)], iidx_v)
    pltpu.sync_copy(j_hbm.at[pl.ds(base, TB)], jidx_v)
    pltpu.sync_copy(kf_hbm.at[pl.ds(base * N_NEG, TB * N_NEG)], kidx_v)

    iota = lax.iota(jnp.int32, L)
    zerof = jnp.zeros((L,), jnp.float32)
    zeroi = jnp.zeros((L,), jnp.int32)

    bufs = ((urows0, irows0, nrows0, sem0),
            (urows1, irows1, nrows1, sem1))

    def fire(c, p):
        urows, irows, nrows, sem = bufs[p]
        pltpu.async_copy(ut_hbm.at[iidx_v.at[pl.ds(c * C, C)]], urows, sem)
        pltpu.async_copy(it_hbm.at[jidx_v.at[pl.ds(c * C, C)]], irows, sem)
        pltpu.async_copy(it_hbm.at[kidx_v.at[pl.ds(c * KC, KC)]], nrows, sem)

    def drain(c, p):
        urows, irows, nrows, sem = bufs[p]
        pltpu.make_async_copy(ut_hbm.at[iidx_v.at[pl.ds(c * C, C)]],
                              urows, sem).wait()
        pltpu.make_async_copy(it_hbm.at[jidx_v.at[pl.ds(c * C, C)]],
                              irows, sem).wait()
        pltpu.make_async_copy(it_hbm.at[kidx_v.at[pl.ds(c * KC, KC)]],
                              nrows, sem).wait()

    def group_body(g, acc, p):
        urows, irows, nrows = bufs[p][0], bufs[p][1], bufs[p][2]
        rowv = g * L + iota

        @pl.loop(0, LATENT_DIM, init_carry=(iota, zerof, zerof, zerof))
        def pass1(d, carry):
            colv, u2, i2, ui = carry
            u = plsc.load_gather(urows, [rowv, colv])
            it = plsc.load_gather(irows, [rowv, colv])
            return ((colv + 1) & (LATENT_DIM - 1),
                    u2 + u * u, i2 + it * it, ui + u * it)

        _, u2, i2, ui = pass1
        su = _clip_scale(u2)
        si = _clip_scale(i2)
        a = MARGIN + si * si * i2 - 2.0 * su * si * ui
        su2 = 2.0 * su

        cnt = zeroi
        pr = zerof
        for h in range(2):
            nrow0 = rowv * N_NEG + h * NHALF
            init = (iota,) + (zerof,) * (2 * NHALF)

            @pl.loop(0, LATENT_DIM, init_carry=init)
            def neg_pass(d, carry):
                colv = carry[0]
                n2s = list(carry[1:1 + NHALF])
                uns = list(carry[1 + NHALF:])
                u = plsc.load_gather(urows, [rowv, colv])
                for n in range(NHALF):
                    x = plsc.load_gather(nrows, [nrow0 + n, colv])
                    n2s[n] = n2s[n] + x * x
                    uns[n] = uns[n] + u * x
                return ((colv + 1) & (LATENT_DIM - 1),) + tuple(n2s) + tuple(uns)

            n2s = neg_pass[1:1 + NHALF]
            uns = neg_pass[1 + NHALF:]
            for n in range(NHALF):
                sn = _clip_scale(n2s[n])
                m = a - sn * (sn * n2s[n] - su2 * uns[n])
                pos = m > 0.0
                cnt = cnt + jnp.where(pos, 1, 0).astype(jnp.int32)
                pr = pr + jnp.where(pos, m, 0.0)

        w = plsc.load_gather(wtab_v, [cnt])
        return acc + w * pr

    def compute(acc, p):
        for g in range(C // L):
            acc = group_body(g, acc, p)
        return acc

    fire(0, 0)

    @pl.loop(0, NCHUNK, step=2, init_carry=zerof)
    def chunk_loop(c, acc):
        fire(c + 1, 1)
        drain(c, 0)
        acc = compute(acc, 0)

        @pl.when(c + 2 < NCHUNK)
        def _():
            fire(c + 2, 0)

        drain(c + 1, 1)
        return compute(acc, 1)

    loss_v[...] = chunk_loop
    pltpu.sync_copy(loss_v, out_hbm.at[pl.ds(wid * L, L)])


NROW = 1000000
SUP = 512
NSUP = NROW // SUP
NT = 122
TAIL = NROW - NSUP * SUP


def _ph1_body(utt, itt, utail, itail, uout, iout, in0, in1, in2, in3,
              ob0, ob1, semi0, semi1, semi2, semi3, semo0, semo1):
    cid = lax.axis_index("c")
    sid = lax.axis_index("s")
    wid = sid * NC + cid
    w16 = wid & 15

    iota = lax.iota(jnp.int32, L)
    inb = (in0, in1, in2, in3)
    ob = (ob0, ob1)
    semi = (semi0, semi1, semi2, semi3)
    semo = (semo0, semo1)

    def transpose_super(p, q):
        @pl.loop(0, SUP // L)
        def _(r0):
            rvec = r0 * L + iota
            rvec32 = rvec * LATENT_DIM
            for c0 in range(LATENT_DIM):
                cvec = (c0 + iota) & (LATENT_DIM - 1)
                v = plsc.load_gather(inb[p], [cvec, rvec])
                plsc.store_scatter(ob[q], [rvec32 + cvec], v)

    def run_table(src, tail, dst):
        def s_of(t):
            return w16 + 16 * t

        def fire_in(t, p):
            off = pl.multiple_of(s_of(t) * SUP, SUP)
            pltpu.async_copy(src.at[:, pl.ds(off, SUP)], inb[p], semi[p])

        def drain_in(t, p):
            off = pl.multiple_of(s_of(t) * SUP, SUP)
            pltpu.make_async_copy(src.at[:, pl.ds(off, SUP)], inb[p],
                                  semi[p]).wait()

        def fire_out(t, q):
            off = pl.multiple_of(s_of(t) * SUP * LATENT_DIM, SUP * LATENT_DIM)
            pltpu.async_copy(ob[q], dst.at[pl.ds(off, SUP * LATENT_DIM)],
                             semo[q])

        def drain_out(t, q):
            off = pl.multiple_of(s_of(t) * SUP * LATENT_DIM, SUP * LATENT_DIM)
            pltpu.make_async_copy(ob[q], dst.at[pl.ds(off, SUP * LATENT_DIM)],
                                  semo[q]).wait()

        def step(t, tq, p, q):
            @pl.when(tq + 3 < NT)
            def _():
                fire_in(tq + 3, (p + 3) % 4)

            drain_in(tq, p)

            @pl.when(tq >= 2)
            def _():
                drain_out(tq - 2, q)

            transpose_super(p, q)
            fire_out(tq, q)

        fire_in(0, 0)
        fire_in(1, 1)
        fire_in(2, 2)

        @pl.loop(0, NT - 2, step=4)
        def _(t):
            for qq in range(4):
                step(t, t + qq, qq, qq % 2)

        step(NT - 2, NT - 2, (NT - 2) % 4, 0)
        step(NT - 1, NT - 1, (NT - 1) % 4, 1)
        drain_out(NT - 2, 0)
        drain_out(NT - 1, 1)

        @pl.when(w16 == 0)
        def _():
            pltpu.sync_copy(src.at[:, pl.ds((NSUP - 1) * SUP, SUP)], in0)
            transpose_super(0, 0)
            pltpu.sync_copy(ob0, dst.at[pl.ds((NSUP - 1) * SUP * LATENT_DIM,
                                              SUP * LATENT_DIM)])

        @pl.when(w16 == 1)
        def _():
            pltpu.sync_copy(tail, ob0.at[pl.ds(0, TAIL * LATENT_DIM)])
            pltpu.sync_copy(ob0.at[pl.ds(0, TAIL * LATENT_DIM)],
                            dst.at[pl.ds(NSUP * SUP * LATENT_DIM,
                                         TAIL * LATENT_DIM)])

    @pl.when(wid < 16)
    def _():
        run_table(utt, utail, uout)

    @pl.when(wid >= 16)
    def _():
        run_table(itt, itail, iout)


@functools.cache
def _ph1_kernel():
    return functools.partial(
        pl.kernel,
        out_type=(jax.ShapeDtypeStruct((NROW * LATENT_DIM,), jnp.float32),
                  jax.ShapeDtypeStruct((NROW * LATENT_DIM,), jnp.float32)),
        mesh=plsc.VectorSubcoreMesh(core_axis_name="c", subcore_axis_name="s",
                                    num_cores=NC, num_subcores=NS),
        scratch_types=[
            pltpu.VMEM((LATENT_DIM, SUP), jnp.float32),
            pltpu.VMEM((LATENT_DIM, SUP), jnp.float32),
            pltpu.VMEM((LATENT_DIM, SUP), jnp.float32),
            pltpu.VMEM((LATENT_DIM, SUP), jnp.float32),
            pltpu.VMEM((SUP * LATENT_DIM,), jnp.float32),
            pltpu.VMEM((SUP * LATENT_DIM,), jnp.float32),
            pltpu.SemaphoreType.DMA,
            pltpu.SemaphoreType.DMA,
            pltpu.SemaphoreType.DMA,
            pltpu.SemaphoreType.DMA,
            pltpu.SemaphoreType.DMA,
            pltpu.SemaphoreType.DMA,
        ],
        compiler_params=pltpu.CompilerParams(needs_layout_passes=False,
                                             use_tc_tiling_on_sc=True),
    )(_ph1_body)


@functools.cache
def _sc_kernel():
    row_scratch = [pltpu.VMEM((C, LATENT_DIM), jnp.float32),
                   pltpu.VMEM((C, LATENT_DIM), jnp.float32),
                   pltpu.VMEM((KC, LATENT_DIM), jnp.float32)]
    return functools.partial(
        pl.kernel,
        out_type=jax.ShapeDtypeStruct((NW * L,), jnp.float32),
        mesh=plsc.VectorSubcoreMesh(core_axis_name="c", subcore_axis_name="s",
                                    num_cores=NC, num_subcores=NS),
        scratch_types=[
            pltpu.VMEM((TB,), jnp.int32),
            pltpu.VMEM((TB,), jnp.int32),
            pltpu.VMEM((TB * N_NEG,), jnp.int32),
        ] + row_scratch + row_scratch + [
            pltpu.VMEM((32,), jnp.float32),
            pltpu.VMEM((L,), jnp.float32),
            pltpu.SemaphoreType.DMA,
            pltpu.SemaphoreType.DMA,
        ],
        compiler_params=pltpu.CompilerParams(needs_layout_passes=False,
                                             use_tc_tiling_on_sc=False),
    )(_sc_body)


@jax.jit
def kernel(i, j, k, user_table, item_table):
    i = i.astype(jnp.int32)
    j = j.astype(jnp.int32)
    kf = k.astype(jnp.int32).reshape(BATCH * N_NEG)
    cnts = jnp.arange(N_NEG + 1, dtype=jnp.float32)
    wtab = jnp.log(N_ITEMS * cnts / N_NEG + 1.0)
    wtab = jnp.concatenate(
        [wtab, jnp.zeros((32 - (N_NEG + 1),), jnp.float32)])
    utail = user_table[NSUP * SUP:].reshape(TAIL * LATENT_DIM)
    itail = item_table[NSUP * SUP:].reshape(TAIL * LATENT_DIM)
    uflat, iflat = _ph1_kernel()(user_table.T, item_table.T, utail, itail)
    uscr = uflat.reshape(NROW, LATENT_DIM)
    iscr = iflat.reshape(NROW, LATENT_DIM)
    partial = _sc_kernel()(i, j, kf, uscr, iscr, wtab)
    return jnp.sum(partial)
